# baseline (speedup 1.0000x reference)
import functools

import jax
import jax.numpy as jnp
from jax import lax
from jax.experimental import pallas as pl
from jax.experimental.pallas import tpu as pltpu
from jax.experimental.pallas import tpu_sc as plsc

N_NODES = 10000
N_FEAT = 128
HID = 128
N_T = 16
N_CLS = 8
NP = 10240
BLK = 512
N_EDGES = 320000
NC = 2
NS = 16
W_CH = 80
CHK = 128
EP = NC * NS * W_CH * CHK
STRIPE = NP // NS

_SC_MESH = dict(core_axis_name="c", subcore_axis_name="s")


def _sc_deg_kernel(dstr, zeros1):
    @functools.partial(
        pl.kernel,
        out_type=jax.ShapeDtypeStruct((NC, NP), jnp.float32),
        mesh=plsc.VectorSubcoreMesh(**_SC_MESH),
        scratch_types=[
            pltpu.VMEM((W_CH, CHK), jnp.int32),
            pltpu.VMEM((CHK,), jnp.float32),
            pltpu.VMEM_SHARED((NP,), jnp.float32),
        ],
    )
    def body(dstr_hbm, zeros_hbm, out_hbm, dst_v, ones_v, deg_sh):
        c = lax.axis_index("c")
        s = lax.axis_index("s")
        pltpu.sync_copy(dstr_hbm.at[c, s], dst_v)
        pltpu.sync_copy(zeros_hbm.at[pl.ds(s * STRIPE, STRIPE)],
                        deg_sh.at[pl.ds(s * STRIPE, STRIPE)])
        for i in range(CHK // 16):
            ones_v[pl.ds(i * 16, 16)] = jnp.ones((16,), jnp.float32)
        plsc.subcore_barrier()

        def step(j, carry):
            pltpu.sync_copy(ones_v, deg_sh.at[dst_v.at[j]], add=True)
            return carry

        lax.fori_loop(0, W_CH, step, 0)
        plsc.subcore_barrier()
        pltpu.sync_copy(deg_sh.at[pl.ds(s * STRIPE, STRIPE)],
                        out_hbm.at[c, pl.ds(s * STRIPE, STRIPE)])

    return body(dstr, zeros1)


def _sc_agg(p, srcr, dstr, zeros, D, tc_tiling=True):
    H = W_CH // 2

    @functools.partial(
        pl.kernel,
        out_type=jax.ShapeDtypeStruct((NC, NP, D), jnp.float32),
        mesh=plsc.VectorSubcoreMesh(**_SC_MESH),
        compiler_params=pltpu.CompilerParams(use_tc_tiling_on_sc=tc_tiling),
        scratch_types=[
            pltpu.VMEM((H + 8, CHK), jnp.int32),
            pltpu.VMEM((H, CHK), jnp.int32),
            pltpu.VMEM((CHK, D), jnp.float32),
            pltpu.VMEM((CHK, D), jnp.float32),
            pltpu.VMEM_SHARED((NP, D), jnp.float32),
            pltpu.SemaphoreType.DMA,
            pltpu.SemaphoreType.DMA,
        ],
    )
    def body(p_hbm, srcr_hbm, dstr_hbm, zeros_hbm, out_hbm,
             src_v, dst_v, rows0, rows1, acc_sh, sem0, sem1):
        c = lax.axis_index("c")
        s = lax.axis_index("s")
        pltpu.sync_copy(zeros_hbm.at[pl.ds(s * STRIPE, STRIPE)],
                        acc_sh.at[pl.ds(s * STRIPE, STRIPE)])
        plsc.subcore_barrier()
        for ph in range(2):
            base = ph * H
            pltpu.sync_copy(srcr_hbm.at[c, s, pl.ds(base, H + 8)], src_v)
            pltpu.sync_copy(dstr_hbm.at[c, s, pl.ds(base, H)], dst_v)
            pltpu.async_copy(p_hbm.at[src_v.at[0]], rows0, sem0)
            pltpu.async_copy(p_hbm.at[src_v.at[1]], rows1, sem1)

            def step(jj, carry):
                j = 2 * jj
                pltpu.make_async_copy(p_hbm.at[src_v.at[0]], rows0, sem0).wait()
                pltpu.sync_copy(rows0, acc_sh.at[dst_v.at[j]], add=True)
                pltpu.async_copy(p_hbm.at[src_v.at[j + 2]], rows0, sem0)
                pltpu.make_async_copy(p_hbm.at[src_v.at[1]], rows1, sem1).wait()
                pltpu.sync_copy(rows1, acc_sh.at[dst_v.at[j + 1]], add=True)
                pltpu.async_copy(p_hbm.at[src_v.at[j + 3]], rows1, sem1)
                return carry

            lax.fori_loop(0, H // 2, step, 0)
            pltpu.make_async_copy(p_hbm.at[src_v.at[0]], rows0, sem0).wait()
            pltpu.make_async_copy(p_hbm.at[src_v.at[1]], rows1, sem1).wait()
        plsc.subcore_barrier()
        pltpu.sync_copy(acc_sh.at[pl.ds(s * STRIPE, STRIPE)],
                        out_hbm.at[c, pl.ds(s * STRIPE, STRIPE)])

    return body(p, srcr, dstr, zeros)


def _tc_mm_body(x_ref, w_ref, out_ref):
    out_ref[...] = jnp.dot(x_ref[...], w_ref[...],
                           preferred_element_type=jnp.float32)


def _tc_mm(x, W):
    return pl.pallas_call(
        _tc_mm_body,
        grid=(NP // BLK,),
        in_specs=[_row_spec(N_FEAT), _full_spec(N_FEAT, HID)],
        out_specs=_row_spec(HID),
        out_shape=jax.ShapeDtypeStruct((NP, HID), jnp.float32),
    )(x, W)


def _tc_a_body(deg_ref, a_ref, p1_ref, dinv_ref, dege_ref):
    deg_e = deg_ref[0, :] + deg_ref[1, :]
    dinv = lax.rsqrt(deg_e + 1.0)
    p1_ref[...] = a_ref[...] * dinv[:, None]
    dinv_ref[...] = dinv[:, None]
    dege_ref[...] = deg_e[:, None]


def _tc_b_body(agg_ref, p_ref, dinv_ref, b_ref, w_ref, out_ref):
    dinv = dinv_ref[...]
    agg = agg_ref[0] + agg_ref[1]
    h = jnp.maximum(dinv * (agg + p_ref[...]) + b_ref[...], 0.0)
    out_ref[...] = jnp.dot(h, w_ref[...], preferred_element_type=jnp.float32) * dinv


def _tc_c_body(agg_ref, p_ref, dinv_ref, dege_ref, b2_ref, w3a_ref, w3b_ref,
               tfeat_ref, tadj_ref, alpha_ref, p3_ref):
    dinv = dinv_ref[...]
    agg = agg_ref[0] + agg_ref[1]
    h2 = jnp.maximum(dinv * (agg + p_ref[...]) + b2_ref[...], 0.0)
    tfeat = tfeat_ref[...]
    tadj = tadj_ref[...]
    t_sq = jnp.mean(jnp.sum(tfeat * tfeat, axis=2), axis=1)
    t_mean = jnp.mean(tfeat, axis=1)
    t_deg = jnp.mean(jnp.sum(tadj, axis=2), axis=1)
    alpha = 1.0 / (1.0 + jnp.exp(-alpha_ref[0, 0]))
    x_sq = jnp.sum(h2 * h2, axis=1)
    cross = lax.dot_general(h2, t_mean, (((1,), (1,)), ((), ())),
                            preferred_element_type=jnp.float32)
    c_feat = x_sq[:, None] + t_sq[None, :] - 2.0 * cross
    dege = dege_ref[...]
    c_struct = (dege - t_deg[None, :]) ** 2
    y = alpha * c_feat + (1.0 - alpha) * c_struct
    u = (jnp.dot(h2, w3a_ref[...], preferred_element_type=jnp.float32)
         + jnp.dot(y, w3b_ref[...], preferred_element_type=jnp.float32))
    p3_ref[...] = jnp.concatenate([u, jnp.zeros_like(u)], axis=1) * dinv


def _tc_d_body(agg_ref, p3_ref, dinv_ref, b3_ref, out_ref):
    agg = agg_ref[0] + agg_ref[1]
    out_ref[...] = dinv_ref[...] * (agg + p3_ref[...]) + b3_ref[...]


def _row_spec(cols):
    return pl.BlockSpec((BLK, cols), lambda i: (i, 0))


def _part_spec(cols):
    return pl.BlockSpec((2, BLK, cols), lambda i: (0, i, 0))


def _full_spec(*shape):
    return pl.BlockSpec(shape, lambda i: (0,) * len(shape))


def _tc_a(deg_parts, a0):
    return pl.pallas_call(
        _tc_a_body,
        grid=(NP // BLK,),
        in_specs=[pl.BlockSpec((2, BLK), lambda i: (0, i)), _row_spec(HID)],
        out_specs=[_row_spec(HID), _row_spec(1), _row_spec(1)],
        out_shape=[jax.ShapeDtypeStruct((NP, HID), jnp.float32),
                   jax.ShapeDtypeStruct((NP, 1), jnp.float32),
                   jax.ShapeDtypeStruct((NP, 1), jnp.float32)],
    )(deg_parts, a0)


def _tc_b(agg, p, dinv, b, W):
    return pl.pallas_call(
        _tc_b_body,
        grid=(NP // BLK,),
        in_specs=[_part_spec(HID), _row_spec(HID), _row_spec(1),
                  _full_spec(1, HID), _full_spec(HID, HID)],
        out_specs=_row_spec(HID),
        out_shape=jax.ShapeDtypeStruct((NP, HID), jnp.float32),
    )(agg, p, dinv, b, W)


def _tc_c(agg, p, dinv, dege, b2, W3a, W3b, T_feat, T_adj, alpha):
    return pl.pallas_call(
        _tc_c_body,
        grid=(NP // BLK,),
        in_specs=[_part_spec(HID), _row_spec(HID), _row_spec(1), _row_spec(1),
                  _full_spec(1, HID), _full_spec(HID, N_CLS),
                  _full_spec(N_T, N_CLS), _full_spec(N_T, 16, HID),
                  _full_spec(N_T, 16, 16), _full_spec(1, 1)],
        out_specs=_row_spec(16),
        out_shape=jax.ShapeDtypeStruct((NP, 16), jnp.float32),
    )(agg, p, dinv, dege, b2, W3a, W3b, T_feat, T_adj, alpha)


def _tc_d(agg, p3, dinv, b3):
    return pl.pallas_call(
        _tc_d_body,
        grid=(NP // BLK,),
        in_specs=[_part_spec(16), _row_spec(16), _row_spec(1), _full_spec(1, 16)],
        out_specs=_row_spec(16),
        out_shape=jax.ShapeDtypeStruct((NP, 16), jnp.float32),
    )(agg, p3, dinv, b3)


def kernel(x, edge_index, W1, b1, W2, b2, W3, b3, T_feat, T_adj, alpha_param):
    src = edge_index[0]
    dst = edge_index[1]
    npad = EP - N_EDGES
    pad_src = (jnp.arange(npad, dtype=jnp.int32) % 256)
    pad_dst = N_NODES + (jnp.arange(npad, dtype=jnp.int32) % 240)
    src3 = jnp.concatenate([src, pad_src]).reshape(NC, NS, W_CH, CHK)
    dummy = jnp.broadcast_to(
        (jnp.arange(CHK, dtype=jnp.int32) * 64) % N_NODES, (NC, NS, 8, CHK))
    srcr = jnp.concatenate([src3, dummy], axis=2)
    dstr = jnp.concatenate([dst, pad_dst]).reshape(NC, NS, W_CH, CHK)

    xp = jnp.zeros((NP, N_FEAT), x.dtype).at[:N_NODES].set(x)
    z1 = jnp.zeros((NP,), jnp.float32)
    z128 = jnp.zeros((NP, HID), jnp.float32)
    z16 = jnp.zeros((NP, 16), jnp.float32)

    deg_parts = _sc_deg_kernel(dstr, z1)
    a0 = _tc_mm(xp, W1)

    p1, dinv, dege = _tc_a(deg_parts, a0)

    agg1 = _sc_agg(p1, srcr, dstr, z128, HID)
    p2 = _tc_b(agg1, p1, dinv, b1.reshape(1, HID), W2)
    agg2 = _sc_agg(p2, srcr, dstr, z128, HID)
    p3 = _tc_c(agg2, p2, dinv, dege, b2.reshape(1, HID),
               W3[:HID], W3[HID:], T_feat, T_adj,
               alpha_param.reshape(1, 1))
    agg3 = _sc_agg(p3, srcr, dstr, z16, 16, tc_tiling=False)
    b3p = jnp.concatenate([b3, jnp.zeros((8,), jnp.float32)]).reshape(1, 16)
    out = _tc_d(agg3, p3, dinv, b3p)
    return out[:N_NODES, :N_CLS]


# speedup vs baseline: 1.0518x; 1.0518x over previous
import functools

import jax
import jax.numpy as jnp
from jax import lax
from jax.experimental import pallas as pl
from jax.experimental.pallas import tpu as pltpu
from jax.experimental.pallas import tpu_sc as plsc

N_NODES = 10000
N_FEAT = 128
HID = 128
N_T = 16
N_CLS = 8
NP = 10240
BLK = 512
N_EDGES = 320000
NC = 2
NS = 16
W_CH = 80
CHK = 128
EP = NC * NS * W_CH * CHK
STRIPE = NP // NS

_SC_MESH = dict(core_axis_name="c", subcore_axis_name="s")


def _sc_deg_kernel(dstr, zeros1):
    @functools.partial(
        pl.kernel,
        out_type=jax.ShapeDtypeStruct((NC, NP), jnp.float32),
        mesh=plsc.VectorSubcoreMesh(**_SC_MESH),
        scratch_types=[
            pltpu.VMEM((W_CH, CHK), jnp.int32),
            pltpu.VMEM((CHK,), jnp.float32),
            pltpu.VMEM_SHARED((NP,), jnp.float32),
        ],
    )
    def body(dstr_hbm, zeros_hbm, out_hbm, dst_v, ones_v, deg_sh):
        c = lax.axis_index("c")
        s = lax.axis_index("s")
        pltpu.sync_copy(dstr_hbm.at[c, s], dst_v)
        pltpu.sync_copy(zeros_hbm.at[pl.ds(s * STRIPE, STRIPE)],
                        deg_sh.at[pl.ds(s * STRIPE, STRIPE)])
        for i in range(CHK // 16):
            ones_v[pl.ds(i * 16, 16)] = jnp.ones((16,), jnp.float32)
        plsc.subcore_barrier()

        def step(j, carry):
            pltpu.sync_copy(ones_v, deg_sh.at[dst_v.at[j]], add=True)
            return carry

        lax.fori_loop(0, W_CH, step, 0)
        plsc.subcore_barrier()
        pltpu.sync_copy(deg_sh.at[pl.ds(s * STRIPE, STRIPE)],
                        out_hbm.at[c, pl.ds(s * STRIPE, STRIPE)])

    return body(dstr, zeros1)


def _sc_agg(p, srcr, dstr, zeros, D):
    H = W_CH // 2

    @functools.partial(
        pl.kernel,
        out_type=jax.ShapeDtypeStruct((NC, NP, D), jnp.float32),
        mesh=plsc.VectorSubcoreMesh(**_SC_MESH),
        scratch_types=[
            pltpu.VMEM((H + 8, CHK), jnp.int32),
            pltpu.VMEM((H, CHK), jnp.int32),
            pltpu.VMEM((CHK, D), jnp.float32),
            pltpu.VMEM((CHK, D), jnp.float32),
            pltpu.VMEM_SHARED((NP, D), jnp.float32),
            pltpu.SemaphoreType.DMA,
            pltpu.SemaphoreType.DMA,
        ],
    )
    def body(p_hbm, srcr_hbm, dstr_hbm, zeros_hbm, out_hbm,
             src_v, dst_v, rows0, rows1, acc_sh, sem0, sem1):
        c = lax.axis_index("c")
        s = lax.axis_index("s")
        pltpu.sync_copy(zeros_hbm.at[pl.ds(s * STRIPE, STRIPE)],
                        acc_sh.at[pl.ds(s * STRIPE, STRIPE)])
        plsc.subcore_barrier()
        for ph in range(2):
            base = ph * H
            pltpu.sync_copy(srcr_hbm.at[c, s, pl.ds(base, H + 8)], src_v)
            pltpu.sync_copy(dstr_hbm.at[c, s, pl.ds(base, H)], dst_v)
            pltpu.async_copy(p_hbm.at[src_v.at[0]], rows0, sem0)
            pltpu.async_copy(p_hbm.at[src_v.at[1]], rows1, sem1)

            def step(jj, carry):
                j = 2 * jj
                pltpu.make_async_copy(p_hbm.at[src_v.at[0]], rows0, sem0).wait()
                pltpu.sync_copy(rows0, acc_sh.at[dst_v.at[j]], add=True)
                pltpu.async_copy(p_hbm.at[src_v.at[j + 2]], rows0, sem0)
                pltpu.make_async_copy(p_hbm.at[src_v.at[1]], rows1, sem1).wait()
                pltpu.sync_copy(rows1, acc_sh.at[dst_v.at[j + 1]], add=True)
                pltpu.async_copy(p_hbm.at[src_v.at[j + 3]], rows1, sem1)
                return carry

            lax.fori_loop(0, H // 2, step, 0)
            pltpu.make_async_copy(p_hbm.at[src_v.at[0]], rows0, sem0).wait()
            pltpu.make_async_copy(p_hbm.at[src_v.at[1]], rows1, sem1).wait()
        plsc.subcore_barrier()
        pltpu.sync_copy(acc_sh.at[pl.ds(s * STRIPE, STRIPE)],
                        out_hbm.at[c, pl.ds(s * STRIPE, STRIPE)])

    return body(p, srcr, dstr, zeros)


WIDE = 1024
NW3 = 12


def _sc_agg16(p, srcw, dstw, zeros):
    D = 16

    @functools.partial(
        pl.kernel,
        out_type=jax.ShapeDtypeStruct((NC, NP, D), jnp.float32),
        mesh=plsc.VectorSubcoreMesh(**_SC_MESH),
        compiler_params=pltpu.CompilerParams(use_tc_tiling_on_sc=False),
        scratch_types=[
            pltpu.VMEM((NW3, WIDE), jnp.int32),
            pltpu.VMEM((NW3 - 2, WIDE), jnp.int32),
            pltpu.VMEM((WIDE, D), jnp.float32),
            pltpu.VMEM((WIDE, D), jnp.float32),
            pltpu.VMEM_SHARED((NP, D), jnp.float32),
            pltpu.SemaphoreType.DMA,
            pltpu.SemaphoreType.DMA,
        ],
    )
    def body(p_hbm, srcw_hbm, dstw_hbm, zeros_hbm, out_hbm,
             src_v, dst_v, rows0, rows1, acc_sh, sem0, sem1):
        c = lax.axis_index("c")
        s = lax.axis_index("s")
        pltpu.sync_copy(srcw_hbm.at[c, s], src_v)
        pltpu.sync_copy(dstw_hbm.at[c, s], dst_v)
        pltpu.sync_copy(zeros_hbm.at[pl.ds(s * STRIPE, STRIPE)],
                        acc_sh.at[pl.ds(s * STRIPE, STRIPE)])
        plsc.subcore_barrier()
        pltpu.async_copy(p_hbm.at[src_v.at[0]], rows0, sem0)
        pltpu.async_copy(p_hbm.at[src_v.at[1]], rows1, sem1)

        def step(jj, carry):
            j = 2 * jj
            pltpu.make_async_copy(p_hbm.at[src_v.at[0]], rows0, sem0).wait()
            pltpu.sync_copy(rows0, acc_sh.at[dst_v.at[j]], add=True)
            pltpu.async_copy(p_hbm.at[src_v.at[j + 2]], rows0, sem0)
            pltpu.make_async_copy(p_hbm.at[src_v.at[1]], rows1, sem1).wait()
            pltpu.sync_copy(rows1, acc_sh.at[dst_v.at[j + 1]], add=True)
            pltpu.async_copy(p_hbm.at[src_v.at[j + 3]], rows1, sem1)
            return carry

        lax.fori_loop(0, (NW3 - 2) // 2, step, 0)
        pltpu.make_async_copy(p_hbm.at[src_v.at[0]], rows0, sem0).wait()
        pltpu.make_async_copy(p_hbm.at[src_v.at[1]], rows1, sem1).wait()
        plsc.subcore_barrier()
        pltpu.sync_copy(acc_sh.at[pl.ds(s * STRIPE, STRIPE)],
                        out_hbm.at[c, pl.ds(s * STRIPE, STRIPE)])

    return body(p, srcw, dstw, zeros)


def _tc_a_body(deg_ref, x_ref, w1_ref, p1_ref, dinv_ref, dege_ref):
    deg_e = deg_ref[0, :] + deg_ref[1, :]
    dinv = lax.rsqrt(deg_e + 1.0)
    h = jnp.dot(x_ref[...], w1_ref[...], preferred_element_type=jnp.float32)
    p1_ref[...] = h * dinv[:, None]
    dinv_ref[...] = dinv[:, None]
    dege_ref[...] = deg_e[:, None]


def _tc_b_body(agg_ref, p_ref, dinv_ref, b_ref, w_ref, out_ref):
    dinv = dinv_ref[...]
    agg = agg_ref[0] + agg_ref[1]
    h = jnp.maximum(dinv * (agg + p_ref[...]) + b_ref[...], 0.0)
    out_ref[...] = jnp.dot(h, w_ref[...], preferred_element_type=jnp.float32) * dinv


def _tc_c_body(agg_ref, p_ref, dinv_ref, dege_ref, b2_ref, w3a_ref, w3b_ref,
               tfeat_ref, tadj_ref, alpha_ref, p3_ref):
    dinv = dinv_ref[...]
    agg = agg_ref[0] + agg_ref[1]
    h2 = jnp.maximum(dinv * (agg + p_ref[...]) + b2_ref[...], 0.0)
    tfeat = tfeat_ref[...]
    tadj = tadj_ref[...]
    t_sq = jnp.mean(jnp.sum(tfeat * tfeat, axis=2), axis=1)
    t_mean = jnp.mean(tfeat, axis=1)
    t_deg = jnp.mean(jnp.sum(tadj, axis=2), axis=1)
    alpha = 1.0 / (1.0 + jnp.exp(-alpha_ref[0, 0]))
    x_sq = jnp.sum(h2 * h2, axis=1)
    cross = lax.dot_general(h2, t_mean, (((1,), (1,)), ((), ())),
                            preferred_element_type=jnp.float32)
    c_feat = x_sq[:, None] + t_sq[None, :] - 2.0 * cross
    dege = dege_ref[...]
    c_struct = (dege - t_deg[None, :]) ** 2
    y = alpha * c_feat + (1.0 - alpha) * c_struct
    u = (jnp.dot(h2, w3a_ref[...], preferred_element_type=jnp.float32)
         + jnp.dot(y, w3b_ref[...], preferred_element_type=jnp.float32))
    p3_ref[...] = jnp.concatenate([u, jnp.zeros_like(u)], axis=1) * dinv


def _tc_d_body(agg_ref, p3_ref, dinv_ref, b3_ref, out_ref):
    agg = agg_ref[0] + agg_ref[1]
    out_ref[...] = dinv_ref[...] * (agg + p3_ref[...]) + b3_ref[...]


def _row_spec(cols):
    return pl.BlockSpec((BLK, cols), lambda i: (i, 0))


def _part_spec(cols):
    return pl.BlockSpec((2, BLK, cols), lambda i: (0, i, 0))


def _full_spec(*shape):
    return pl.BlockSpec(shape, lambda i: (0,) * len(shape))


def _tc_a(deg_parts, x, W1):
    return pl.pallas_call(
        _tc_a_body,
        grid=(NP // BLK,),
        in_specs=[pl.BlockSpec((2, BLK), lambda i: (0, i)),
                  _row_spec(N_FEAT), _full_spec(N_FEAT, HID)],
        out_specs=[_row_spec(HID), _row_spec(1), _row_spec(1)],
        out_shape=[jax.ShapeDtypeStruct((NP, HID), jnp.float32),
                   jax.ShapeDtypeStruct((NP, 1), jnp.float32),
                   jax.ShapeDtypeStruct((NP, 1), jnp.float32)],
    )(deg_parts, x, W1)


def _tc_b(agg, p, dinv, b, W):
    return pl.pallas_call(
        _tc_b_body,
        grid=(NP // BLK,),
        in_specs=[_part_spec(HID), _row_spec(HID), _row_spec(1),
                  _full_spec(1, HID), _full_spec(HID, HID)],
        out_specs=_row_spec(HID),
        out_shape=jax.ShapeDtypeStruct((NP, HID), jnp.float32),
    )(agg, p, dinv, b, W)


def _tc_c(agg, p, dinv, dege, b2, W3a, W3b, T_feat, T_adj, alpha):
    return pl.pallas_call(
        _tc_c_body,
        grid=(NP // BLK,),
        in_specs=[_part_spec(HID), _row_spec(HID), _row_spec(1), _row_spec(1),
                  _full_spec(1, HID), _full_spec(HID, N_CLS),
                  _full_spec(N_T, N_CLS), _full_spec(N_T, 16, HID),
                  _full_spec(N_T, 16, 16), _full_spec(1, 1)],
        out_specs=_row_spec(16),
        out_shape=jax.ShapeDtypeStruct((NP, 16), jnp.float32),
    )(agg, p, dinv, dege, b2, W3a, W3b, T_feat, T_adj, alpha)


def _tc_d(agg, p3, dinv, b3):
    return pl.pallas_call(
        _tc_d_body,
        grid=(NP // BLK,),
        in_specs=[_part_spec(16), _row_spec(16), _row_spec(1), _full_spec(1, 16)],
        out_specs=_row_spec(16),
        out_shape=jax.ShapeDtypeStruct((NP, 16), jnp.float32),
    )(agg, p3, dinv, b3)


def kernel(x, edge_index, W1, b1, W2, b2, W3, b3, T_feat, T_adj, alpha_param):
    src = edge_index[0]
    dst = edge_index[1]
    npad = EP - N_EDGES
    pad_src = (jnp.arange(npad, dtype=jnp.int32) % 256)
    pad_dst = N_NODES + (jnp.arange(npad, dtype=jnp.int32) % 240)
    src3 = jnp.concatenate([src, pad_src]).reshape(NC, NS, W_CH, CHK)
    dummy = jnp.broadcast_to(
        (jnp.arange(CHK, dtype=jnp.int32) * 64) % N_NODES, (NC, NS, 16, CHK))
    srcr = jnp.concatenate([src3, dummy], axis=2)
    dstr = jnp.concatenate([dst, pad_dst]).reshape(NC, NS, W_CH, CHK)
    srcw = srcr.reshape(NC, NS, NW3, WIDE)
    dstw = dstr.reshape(NC, NS, NW3 - 2, WIDE)

    xp = jnp.zeros((NP, N_FEAT), x.dtype).at[:N_NODES].set(x)
    z1 = jnp.zeros((NP,), jnp.float32)
    z128 = jnp.zeros((NP, HID), jnp.float32)
    z16 = jnp.zeros((NP, 16), jnp.float32)

    deg_parts = _sc_deg_kernel(dstr, z1)

    p1, dinv, dege = _tc_a(deg_parts, xp, W1)

    agg1 = _sc_agg(p1, srcr, dstr, z128, HID)
    p2 = _tc_b(agg1, p1, dinv, b1.reshape(1, HID), W2)
    agg2 = _sc_agg(p2, srcr, dstr, z128, HID)
    p3 = _tc_c(agg2, p2, dinv, dege, b2.reshape(1, HID),
               W3[:HID], W3[HID:], T_feat, T_adj,
               alpha_param.reshape(1, 1))
    agg3 = _sc_agg16(p3, srcw, dstw, z16)
    b3p = jnp.concatenate([b3, jnp.zeros((8,), jnp.float32)]).reshape(1, 16)
    out = _tc_d(agg3, p3, dinv, b3p)
    return out[:N_NODES, :N_CLS]


# speedup vs baseline: 1.0598x; 1.0076x over previous
import functools

import jax
import jax.numpy as jnp
from jax import lax
from jax.experimental import pallas as pl
from jax.experimental.pallas import tpu as pltpu
from jax.experimental.pallas import tpu_sc as plsc

N_NODES = 10000
N_FEAT = 128
HID = 128
N_T = 16
N_CLS = 8
NP = 10240
BLK = 512
N_EDGES = 320000
NC = 2
NS = 16
W_CH = 80
CHK = 128
EP = NC * NS * W_CH * CHK
STRIPE = NP // NS

_SC_MESH = dict(core_axis_name="c", subcore_axis_name="s")
WIDE = 1024
NW3 = 12


def _sc_deg_kernel(dstw, zeros1):
    @functools.partial(
        pl.kernel,
        out_type=jax.ShapeDtypeStruct((NC, NP), jnp.float32),
        mesh=plsc.VectorSubcoreMesh(**_SC_MESH),
        compiler_params=pltpu.CompilerParams(use_tc_tiling_on_sc=False),
        scratch_types=[
            pltpu.VMEM((NW3 - 2, WIDE), jnp.int32),
            pltpu.VMEM((WIDE,), jnp.float32),
            pltpu.VMEM_SHARED((NP,), jnp.float32),
        ],
    )
    def body(dstw_hbm, zeros_hbm, out_hbm, dst_v, ones_v, deg_sh):
        c = lax.axis_index("c")
        s = lax.axis_index("s")
        pltpu.sync_copy(dstw_hbm.at[c, s], dst_v)
        pltpu.sync_copy(zeros_hbm.at[pl.ds(s * STRIPE, STRIPE)],
                        deg_sh.at[pl.ds(s * STRIPE, STRIPE)])
        for i in range(WIDE // 16):
            ones_v[pl.ds(i * 16, 16)] = jnp.ones((16,), jnp.float32)
        plsc.subcore_barrier()

        def step(j, carry):
            pltpu.sync_copy(ones_v, deg_sh.at[dst_v.at[j]], add=True)
            return carry

        lax.fori_loop(0, NW3 - 2, step, 0)
        plsc.subcore_barrier()
        pltpu.sync_copy(deg_sh.at[pl.ds(s * STRIPE, STRIPE)],
                        out_hbm.at[c, pl.ds(s * STRIPE, STRIPE)])

    return body(dstw, zeros1)


def _sc_agg(p, srcr, dstr, zeros, D):
    H = W_CH // 2

    @functools.partial(
        pl.kernel,
        out_type=jax.ShapeDtypeStruct((NC, NP, D), jnp.float32),
        mesh=plsc.VectorSubcoreMesh(**_SC_MESH),
        scratch_types=[
            pltpu.VMEM((H + 8, CHK), jnp.int32),
            pltpu.VMEM((H, CHK), jnp.int32),
            pltpu.VMEM((CHK, D), jnp.float32),
            pltpu.VMEM((CHK, D), jnp.float32),
            pltpu.VMEM_SHARED((NP, D), jnp.float32),
            pltpu.SemaphoreType.DMA,
            pltpu.SemaphoreType.DMA,
        ],
    )
    def body(p_hbm, srcr_hbm, dstr_hbm, zeros_hbm, out_hbm,
             src_v, dst_v, rows0, rows1, acc_sh, sem0, sem1):
        c = lax.axis_index("c")
        s = lax.axis_index("s")
        pltpu.sync_copy(zeros_hbm.at[pl.ds(s * STRIPE, STRIPE)],
                        acc_sh.at[pl.ds(s * STRIPE, STRIPE)])
        plsc.subcore_barrier()
        for ph in range(2):
            base = ph * H
            pltpu.sync_copy(srcr_hbm.at[c, s, pl.ds(base, H + 8)], src_v)
            pltpu.sync_copy(dstr_hbm.at[c, s, pl.ds(base, H)], dst_v)
            pltpu.async_copy(p_hbm.at[src_v.at[0]], rows0, sem0)
            pltpu.async_copy(p_hbm.at[src_v.at[1]], rows1, sem1)

            def step(jj, carry):
                j = 2 * jj
                pltpu.make_async_copy(p_hbm.at[src_v.at[0]], rows0, sem0).wait()
                pltpu.sync_copy(rows0, acc_sh.at[dst_v.at[j]], add=True)
                pltpu.async_copy(p_hbm.at[src_v.at[j + 2]], rows0, sem0)
                pltpu.make_async_copy(p_hbm.at[src_v.at[1]], rows1, sem1).wait()
                pltpu.sync_copy(rows1, acc_sh.at[dst_v.at[j + 1]], add=True)
                pltpu.async_copy(p_hbm.at[src_v.at[j + 3]], rows1, sem1)
                return carry

            lax.fori_loop(0, H // 2, step, 0)
            pltpu.make_async_copy(p_hbm.at[src_v.at[0]], rows0, sem0).wait()
            pltpu.make_async_copy(p_hbm.at[src_v.at[1]], rows1, sem1).wait()
        plsc.subcore_barrier()
        pltpu.sync_copy(acc_sh.at[pl.ds(s * STRIPE, STRIPE)],
                        out_hbm.at[c, pl.ds(s * STRIPE, STRIPE)])

    return body(p, srcr, dstr, zeros)


def _sc_agg16(p, srcw, dstw, zeros):
    D = 16

    @functools.partial(
        pl.kernel,
        out_type=jax.ShapeDtypeStruct((NC, NP, D), jnp.float32),
        mesh=plsc.VectorSubcoreMesh(**_SC_MESH),
        compiler_params=pltpu.CompilerParams(use_tc_tiling_on_sc=False),
        scratch_types=[
            pltpu.VMEM((NW3, WIDE), jnp.int32),
            pltpu.VMEM((NW3 - 2, WIDE), jnp.int32),
            pltpu.VMEM((WIDE, D), jnp.float32),
            pltpu.VMEM((WIDE, D), jnp.float32),
            pltpu.VMEM_SHARED((NP, D), jnp.float32),
            pltpu.SemaphoreType.DMA,
            pltpu.SemaphoreType.DMA,
        ],
    )
    def body(p_hbm, srcw_hbm, dstw_hbm, zeros_hbm, out_hbm,
             src_v, dst_v, rows0, rows1, acc_sh, sem0, sem1):
        c = lax.axis_index("c")
        s = lax.axis_index("s")
        pltpu.sync_copy(srcw_hbm.at[c, s], src_v)
        pltpu.sync_copy(dstw_hbm.at[c, s], dst_v)
        pltpu.sync_copy(zeros_hbm.at[pl.ds(s * STRIPE, STRIPE)],
                        acc_sh.at[pl.ds(s * STRIPE, STRIPE)])
        plsc.subcore_barrier()
        pltpu.async_copy(p_hbm.at[src_v.at[0]], rows0, sem0)
        pltpu.async_copy(p_hbm.at[src_v.at[1]], rows1, sem1)

        def step(jj, carry):
            j = 2 * jj
            pltpu.make_async_copy(p_hbm.at[src_v.at[0]], rows0, sem0).wait()
            pltpu.sync_copy(rows0, acc_sh.at[dst_v.at[j]], add=True)
            pltpu.async_copy(p_hbm.at[src_v.at[j + 2]], rows0, sem0)
            pltpu.make_async_copy(p_hbm.at[src_v.at[1]], rows1, sem1).wait()
            pltpu.sync_copy(rows1, acc_sh.at[dst_v.at[j + 1]], add=True)
            pltpu.async_copy(p_hbm.at[src_v.at[j + 3]], rows1, sem1)
            return carry

        lax.fori_loop(0, (NW3 - 2) // 2, step, 0)
        pltpu.make_async_copy(p_hbm.at[src_v.at[0]], rows0, sem0).wait()
        pltpu.make_async_copy(p_hbm.at[src_v.at[1]], rows1, sem1).wait()
        plsc.subcore_barrier()
        pltpu.sync_copy(acc_sh.at[pl.ds(s * STRIPE, STRIPE)],
                        out_hbm.at[c, pl.ds(s * STRIPE, STRIPE)])

    return body(p, srcw, dstw, zeros)


def _tc_a_body(deg_ref, x_ref, w1_ref, p1_ref, dinv_ref, dege_ref):
    deg_e = deg_ref[0, :] + deg_ref[1, :]
    dinv = lax.rsqrt(deg_e + 1.0)
    h = jnp.dot(x_ref[...], w1_ref[...], preferred_element_type=jnp.float32)
    p1_ref[...] = h * dinv[:, None]
    dinv_ref[...] = dinv[:, None]
    dege_ref[...] = deg_e[:, None]


def _tc_b_body(agg_ref, p_ref, dinv_ref, b_ref, w_ref, out_ref):
    dinv = dinv_ref[...]
    agg = agg_ref[0] + agg_ref[1]
    h = jnp.maximum(dinv * (agg + p_ref[...]) + b_ref[...], 0.0)
    out_ref[...] = jnp.dot(h, w_ref[...], preferred_element_type=jnp.float32) * dinv


def _tc_c_body(agg_ref, p_ref, dinv_ref, dege_ref, b2_ref, w3a_ref, w3b_ref,
               tfeat_ref, tadj_ref, alpha_ref, p3_ref):
    dinv = dinv_ref[...]
    agg = agg_ref[0] + agg_ref[1]
    h2 = jnp.maximum(dinv * (agg + p_ref[...]) + b2_ref[...], 0.0)
    tfeat = tfeat_ref[...]
    tadj = tadj_ref[...]
    t_sq = jnp.mean(jnp.sum(tfeat * tfeat, axis=2), axis=1)
    t_mean = jnp.mean(tfeat, axis=1)
    t_deg = jnp.mean(jnp.sum(tadj, axis=2), axis=1)
    alpha = 1.0 / (1.0 + jnp.exp(-alpha_ref[0, 0]))
    x_sq = jnp.sum(h2 * h2, axis=1)
    cross = lax.dot_general(h2, t_mean, (((1,), (1,)), ((), ())),
                            preferred_element_type=jnp.float32)
    c_feat = x_sq[:, None] + t_sq[None, :] - 2.0 * cross
    dege = dege_ref[...]
    c_struct = (dege - t_deg[None, :]) ** 2
    y = alpha * c_feat + (1.0 - alpha) * c_struct
    u = (jnp.dot(h2, w3a_ref[...], preferred_element_type=jnp.float32)
         + jnp.dot(y, w3b_ref[...], preferred_element_type=jnp.float32))
    p3_ref[...] = jnp.concatenate([u, jnp.zeros_like(u)], axis=1) * dinv


def _tc_d_body(agg_ref, p3_ref, dinv_ref, b3_ref, out_ref):
    agg = agg_ref[0] + agg_ref[1]
    out_ref[...] = dinv_ref[...] * (agg + p3_ref[...]) + b3_ref[...]


def _row_spec(cols):
    return pl.BlockSpec((BLK, cols), lambda i: (i, 0))


def _part_spec(cols):
    return pl.BlockSpec((2, BLK, cols), lambda i: (0, i, 0))


def _full_spec(*shape):
    return pl.BlockSpec(shape, lambda i: (0,) * len(shape))


def _tc_a(deg_parts, x, W1):
    return pl.pallas_call(
        _tc_a_body,
        grid=(NP // BLK,),
        in_specs=[pl.BlockSpec((2, BLK), lambda i: (0, i)),
                  _row_spec(N_FEAT), _full_spec(N_FEAT, HID)],
        out_specs=[_row_spec(HID), _row_spec(1), _row_spec(1)],
        out_shape=[jax.ShapeDtypeStruct((NP, HID), jnp.float32),
                   jax.ShapeDtypeStruct((NP, 1), jnp.float32),
                   jax.ShapeDtypeStruct((NP, 1), jnp.float32)],
    )(deg_parts, x, W1)


def _tc_b(agg, p, dinv, b, W):
    return pl.pallas_call(
        _tc_b_body,
        grid=(NP // BLK,),
        in_specs=[_part_spec(HID), _row_spec(HID), _row_spec(1),
                  _full_spec(1, HID), _full_spec(HID, HID)],
        out_specs=_row_spec(HID),
        out_shape=jax.ShapeDtypeStruct((NP, HID), jnp.float32),
    )(agg, p, dinv, b, W)


def _tc_c(agg, p, dinv, dege, b2, W3a, W3b, T_feat, T_adj, alpha):
    return pl.pallas_call(
        _tc_c_body,
        grid=(NP // BLK,),
        in_specs=[_part_spec(HID), _row_spec(HID), _row_spec(1), _row_spec(1),
                  _full_spec(1, HID), _full_spec(HID, N_CLS),
                  _full_spec(N_T, N_CLS), _full_spec(N_T, 16, HID),
                  _full_spec(N_T, 16, 16), _full_spec(1, 1)],
        out_specs=_row_spec(16),
        out_shape=jax.ShapeDtypeStruct((NP, 16), jnp.float32),
    )(agg, p, dinv, dege, b2, W3a, W3b, T_feat, T_adj, alpha)


def _tc_d(agg, p3, dinv, b3):
    return pl.pallas_call(
        _tc_d_body,
        grid=(NP // BLK,),
        in_specs=[_part_spec(16), _row_spec(16), _row_spec(1), _full_spec(1, 16)],
        out_specs=_row_spec(16),
        out_shape=jax.ShapeDtypeStruct((NP, 16), jnp.float32),
    )(agg, p3, dinv, b3)


def kernel(x, edge_index, W1, b1, W2, b2, W3, b3, T_feat, T_adj, alpha_param):
    src = edge_index[0]
    dst = edge_index[1]
    npad = EP - N_EDGES
    pad_src = (jnp.arange(npad, dtype=jnp.int32) % 256)
    pad_dst = N_NODES + (jnp.arange(npad, dtype=jnp.int32) % 240)
    src3 = jnp.concatenate([src, pad_src]).reshape(NC, NS, W_CH, CHK)
    dummy = jnp.broadcast_to(
        (jnp.arange(CHK, dtype=jnp.int32) * 64) % N_NODES, (NC, NS, 16, CHK))
    srcr = jnp.concatenate([src3, dummy], axis=2)
    dstr = jnp.concatenate([dst, pad_dst]).reshape(NC, NS, W_CH, CHK)
    srcw = srcr.reshape(NC, NS, NW3, WIDE)
    dstw = dstr.reshape(NC, NS, NW3 - 2, WIDE)

    xp = jnp.zeros((NP, N_FEAT), x.dtype).at[:N_NODES].set(x)
    z1 = jnp.zeros((NP,), jnp.float32)
    z128 = jnp.zeros((NP, HID), jnp.float32)
    z16 = jnp.zeros((NP, 16), jnp.float32)

    deg_parts = _sc_deg_kernel(dstw, z1)

    p1, dinv, dege = _tc_a(deg_parts, xp, W1)

    agg1 = _sc_agg(p1, srcr, dstr, z128, HID)
    p2 = _tc_b(agg1, p1, dinv, b1.reshape(1, HID), W2)
    agg2 = _sc_agg(p2, srcr, dstr, z128, HID)
    p3 = _tc_c(agg2, p2, dinv, dege, b2.reshape(1, HID),
               W3[:HID], W3[HID:], T_feat, T_adj,
               alpha_param.reshape(1, 1))
    agg3 = _sc_agg16(p3, srcw, dstw, z16)
    b3p = jnp.concatenate([b3, jnp.zeros((8,), jnp.float32)]).reshape(1, 16)
    out = _tc_d(agg3, p3, dinv, b3p)
    return out[:N_NODES, :N_CLS]


# speedup vs baseline: 1.1208x; 1.0575x over previous
import functools

import jax
import jax.numpy as jnp
from jax import lax
from jax.experimental import pallas as pl
from jax.experimental.pallas import tpu as pltpu
from jax.experimental.pallas import tpu_sc as plsc

N_NODES = 10000
N_FEAT = 128
HID = 128
N_T = 16
N_CLS = 8
NP = 10240
BLK = 1024
N_EDGES = 320000
NC = 2
NS = 16
W_CH = 80
CHK = 128
EP = NC * NS * W_CH * CHK
STRIPE = NP // NS

_SC_MESH = dict(core_axis_name="c", subcore_axis_name="s")
WIDE = 1024
NW3 = 12


def _sc_deg_kernel(dstw, zeros1):
    @functools.partial(
        pl.kernel,
        out_type=jax.ShapeDtypeStruct((NC, NP), jnp.float32),
        mesh=plsc.VectorSubcoreMesh(**_SC_MESH),
        compiler_params=pltpu.CompilerParams(use_tc_tiling_on_sc=False),
        scratch_types=[
            pltpu.VMEM((NW3 - 2, WIDE), jnp.int32),
            pltpu.VMEM((WIDE,), jnp.float32),
            pltpu.VMEM_SHARED((NP,), jnp.float32),
        ],
    )
    def body(dstw_hbm, zeros_hbm, out_hbm, dst_v, ones_v, deg_sh):
        c = lax.axis_index("c")
        s = lax.axis_index("s")
        pltpu.sync_copy(dstw_hbm.at[c, s], dst_v)
        pltpu.sync_copy(zeros_hbm.at[pl.ds(s * STRIPE, STRIPE)],
                        deg_sh.at[pl.ds(s * STRIPE, STRIPE)])
        for i in range(WIDE // 16):
            ones_v[pl.ds(i * 16, 16)] = jnp.ones((16,), jnp.float32)
        plsc.subcore_barrier()

        def step(j, carry):
            pltpu.sync_copy(ones_v, deg_sh.at[dst_v.at[j]], add=True)
            return carry

        lax.fori_loop(0, NW3 - 2, step, 0)
        plsc.subcore_barrier()
        pltpu.sync_copy(deg_sh.at[pl.ds(s * STRIPE, STRIPE)],
                        out_hbm.at[c, pl.ds(s * STRIPE, STRIPE)])

    return body(dstw, zeros1)


def _sc_agg(p, srcr, dstr, zeros, D):
    H = W_CH // 2

    @functools.partial(
        pl.kernel,
        out_type=jax.ShapeDtypeStruct((NC, NP, D), jnp.float32),
        mesh=plsc.VectorSubcoreMesh(**_SC_MESH),
        scratch_types=[
            pltpu.VMEM((H + 8, CHK), jnp.int32),
            pltpu.VMEM((H, CHK), jnp.int32),
            pltpu.VMEM((CHK, D), jnp.float32),
            pltpu.VMEM((CHK, D), jnp.float32),
            pltpu.VMEM_SHARED((NP, D), jnp.float32),
            pltpu.SemaphoreType.DMA,
            pltpu.SemaphoreType.DMA,
        ],
    )
    def body(p_hbm, srcr_hbm, dstr_hbm, zeros_hbm, out_hbm,
             src_v, dst_v, rows0, rows1, acc_sh, sem0, sem1):
        c = lax.axis_index("c")
        s = lax.axis_index("s")
        pltpu.sync_copy(zeros_hbm.at[pl.ds(s * STRIPE, STRIPE)],
                        acc_sh.at[pl.ds(s * STRIPE, STRIPE)])
        plsc.subcore_barrier()
        for ph in range(2):
            base = ph * H
            pltpu.sync_copy(srcr_hbm.at[c, s, pl.ds(base, H + 8)], src_v)
            pltpu.sync_copy(dstr_hbm.at[c, s, pl.ds(base, H)], dst_v)
            pltpu.async_copy(p_hbm.at[src_v.at[0]], rows0, sem0)
            pltpu.async_copy(p_hbm.at[src_v.at[1]], rows1, sem1)

            def step(jj, carry):
                j = 2 * jj
                pltpu.make_async_copy(p_hbm.at[src_v.at[0]], rows0, sem0).wait()
                pltpu.sync_copy(rows0, acc_sh.at[dst_v.at[j]], add=True)
                pltpu.async_copy(p_hbm.at[src_v.at[j + 2]], rows0, sem0)
                pltpu.make_async_copy(p_hbm.at[src_v.at[1]], rows1, sem1).wait()
                pltpu.sync_copy(rows1, acc_sh.at[dst_v.at[j + 1]], add=True)
                pltpu.async_copy(p_hbm.at[src_v.at[j + 3]], rows1, sem1)
                return carry

            lax.fori_loop(0, H // 2, step, 0)
            pltpu.make_async_copy(p_hbm.at[src_v.at[0]], rows0, sem0).wait()
            pltpu.make_async_copy(p_hbm.at[src_v.at[1]], rows1, sem1).wait()
        plsc.subcore_barrier()
        pltpu.sync_copy(acc_sh.at[pl.ds(s * STRIPE, STRIPE)],
                        out_hbm.at[c, pl.ds(s * STRIPE, STRIPE)])

    return body(p, srcr, dstr, zeros)


def _sc_agg16(p, srcw, dstw, zeros):
    D = 16

    @functools.partial(
        pl.kernel,
        out_type=jax.ShapeDtypeStruct((NC, NP, D), jnp.float32),
        mesh=plsc.VectorSubcoreMesh(**_SC_MESH),
        compiler_params=pltpu.CompilerParams(use_tc_tiling_on_sc=False),
        scratch_types=[
            pltpu.VMEM((NW3, WIDE), jnp.int32),
            pltpu.VMEM((NW3 - 2, WIDE), jnp.int32),
            pltpu.VMEM((WIDE, D), jnp.float32),
            pltpu.VMEM((WIDE, D), jnp.float32),
            pltpu.VMEM_SHARED((NP, D), jnp.float32),
            pltpu.SemaphoreType.DMA,
            pltpu.SemaphoreType.DMA,
        ],
    )
    def body(p_hbm, srcw_hbm, dstw_hbm, zeros_hbm, out_hbm,
             src_v, dst_v, rows0, rows1, acc_sh, sem0, sem1):
        c = lax.axis_index("c")
        s = lax.axis_index("s")
        pltpu.sync_copy(srcw_hbm.at[c, s], src_v)
        pltpu.sync_copy(dstw_hbm.at[c, s], dst_v)
        pltpu.sync_copy(zeros_hbm.at[pl.ds(s * STRIPE, STRIPE)],
                        acc_sh.at[pl.ds(s * STRIPE, STRIPE)])
        plsc.subcore_barrier()
        pltpu.async_copy(p_hbm.at[src_v.at[0]], rows0, sem0)
        pltpu.async_copy(p_hbm.at[src_v.at[1]], rows1, sem1)

        def step(jj, carry):
            j = 2 * jj
            pltpu.make_async_copy(p_hbm.at[src_v.at[0]], rows0, sem0).wait()
            pltpu.sync_copy(rows0, acc_sh.at[dst_v.at[j]], add=True)
            pltpu.async_copy(p_hbm.at[src_v.at[j + 2]], rows0, sem0)
            pltpu.make_async_copy(p_hbm.at[src_v.at[1]], rows1, sem1).wait()
            pltpu.sync_copy(rows1, acc_sh.at[dst_v.at[j + 1]], add=True)
            pltpu.async_copy(p_hbm.at[src_v.at[j + 3]], rows1, sem1)
            return carry

        lax.fori_loop(0, (NW3 - 2) // 2, step, 0)
        pltpu.make_async_copy(p_hbm.at[src_v.at[0]], rows0, sem0).wait()
        pltpu.make_async_copy(p_hbm.at[src_v.at[1]], rows1, sem1).wait()
        plsc.subcore_barrier()
        pltpu.sync_copy(acc_sh.at[pl.ds(s * STRIPE, STRIPE)],
                        out_hbm.at[c, pl.ds(s * STRIPE, STRIPE)])

    return body(p, srcw, dstw, zeros)


def _tc_a_body(deg_ref, x_ref, w1_ref, p1_ref, dinv_ref, dege_ref):
    deg_e = deg_ref[0, :] + deg_ref[1, :]
    dinv = lax.rsqrt(deg_e + 1.0)
    h = jnp.dot(x_ref[...], w1_ref[...], preferred_element_type=jnp.float32)
    p1_ref[...] = h * dinv[:, None]
    dinv_ref[...] = dinv[:, None]
    dege_ref[...] = deg_e[:, None]


def _tc_b_body(agg_ref, p_ref, dinv_ref, b_ref, w_ref, out_ref):
    dinv = dinv_ref[...]
    agg = agg_ref[0] + agg_ref[1]
    h = jnp.maximum(dinv * (agg + p_ref[...]) + b_ref[...], 0.0)
    out_ref[...] = jnp.dot(h, w_ref[...], preferred_element_type=jnp.float32) * dinv


def _tc_c_body(agg_ref, p_ref, dinv_ref, dege_ref, b2_ref, w3a_ref, w3b_ref,
               tfeat_ref, tadj_ref, alpha_ref, p3_ref):
    dinv = dinv_ref[...]
    agg = agg_ref[0] + agg_ref[1]
    h2 = jnp.maximum(dinv * (agg + p_ref[...]) + b2_ref[...], 0.0)
    tfeat = tfeat_ref[...]
    tadj = tadj_ref[...]
    t_sq = jnp.mean(jnp.sum(tfeat * tfeat, axis=2), axis=1)
    t_mean = jnp.mean(tfeat, axis=1)
    t_deg = jnp.mean(jnp.sum(tadj, axis=2), axis=1)
    alpha = 1.0 / (1.0 + jnp.exp(-alpha_ref[0, 0]))
    x_sq = jnp.sum(h2 * h2, axis=1)
    cross = lax.dot_general(h2, t_mean, (((1,), (1,)), ((), ())),
                            preferred_element_type=jnp.float32)
    c_feat = x_sq[:, None] + t_sq[None, :] - 2.0 * cross
    dege = dege_ref[...]
    c_struct = (dege - t_deg[None, :]) ** 2
    y = alpha * c_feat + (1.0 - alpha) * c_struct
    u = (jnp.dot(h2, w3a_ref[...], preferred_element_type=jnp.float32)
         + jnp.dot(y, w3b_ref[...], preferred_element_type=jnp.float32))
    p3_ref[...] = jnp.concatenate([u, jnp.zeros_like(u)], axis=1) * dinv


def _tc_d_body(agg_ref, p3_ref, dinv_ref, b3_ref, out_ref):
    agg = agg_ref[0] + agg_ref[1]
    out_ref[...] = dinv_ref[...] * (agg + p3_ref[...]) + b3_ref[...]


def _row_spec(cols):
    return pl.BlockSpec((BLK, cols), lambda i: (i, 0))


def _part_spec(cols):
    return pl.BlockSpec((2, BLK, cols), lambda i: (0, i, 0))


def _full_spec(*shape):
    return pl.BlockSpec(shape, lambda i: (0,) * len(shape))


def _tc_a(deg_parts, x, W1):
    return pl.pallas_call(
        _tc_a_body,
        grid=(NP // BLK,),
        in_specs=[pl.BlockSpec((2, BLK), lambda i: (0, i)),
                  _row_spec(N_FEAT), _full_spec(N_FEAT, HID)],
        out_specs=[_row_spec(HID), _row_spec(1), _row_spec(1)],
        out_shape=[jax.ShapeDtypeStruct((NP, HID), jnp.float32),
                   jax.ShapeDtypeStruct((NP, 1), jnp.float32),
                   jax.ShapeDtypeStruct((NP, 1), jnp.float32)],
    )(deg_parts, x, W1)


def _tc_b(agg, p, dinv, b, W):
    return pl.pallas_call(
        _tc_b_body,
        grid=(NP // BLK,),
        in_specs=[_part_spec(HID), _row_spec(HID), _row_spec(1),
                  _full_spec(1, HID), _full_spec(HID, HID)],
        out_specs=_row_spec(HID),
        out_shape=jax.ShapeDtypeStruct((NP, HID), jnp.float32),
    )(agg, p, dinv, b, W)


def _tc_c(agg, p, dinv, dege, b2, W3a, W3b, T_feat, T_adj, alpha):
    return pl.pallas_call(
        _tc_c_body,
        grid=(NP // BLK,),
        in_specs=[_part_spec(HID), _row_spec(HID), _row_spec(1), _row_spec(1),
                  _full_spec(1, HID), _full_spec(HID, N_CLS),
                  _full_spec(N_T, N_CLS), _full_spec(N_T, 16, HID),
                  _full_spec(N_T, 16, 16), _full_spec(1, 1)],
        out_specs=_row_spec(16),
        out_shape=jax.ShapeDtypeStruct((NP, 16), jnp.float32),
    )(agg, p, dinv, dege, b2, W3a, W3b, T_feat, T_adj, alpha)


def _tc_d(agg, p3, dinv, b3):
    return pl.pallas_call(
        _tc_d_body,
        grid=(NP // BLK,),
        in_specs=[_part_spec(16), _row_spec(16), _row_spec(1), _full_spec(1, 16)],
        out_specs=_row_spec(16),
        out_shape=jax.ShapeDtypeStruct((NP, 16), jnp.float32),
    )(agg, p3, dinv, b3)


def kernel(x, edge_index, W1, b1, W2, b2, W3, b3, T_feat, T_adj, alpha_param):
    src = edge_index[0]
    dst = edge_index[1]
    npad = EP - N_EDGES
    pad_src = (jnp.arange(npad, dtype=jnp.int32) % 256)
    pad_dst = N_NODES + (jnp.arange(npad, dtype=jnp.int32) % 240)
    src3 = jnp.concatenate([src, pad_src]).reshape(NC, NS, W_CH, CHK)
    dummy = jnp.broadcast_to(
        (jnp.arange(CHK, dtype=jnp.int32) * 64) % N_NODES, (NC, NS, 16, CHK))
    srcr = jnp.concatenate([src3, dummy], axis=2)
    dstr = jnp.concatenate([dst, pad_dst]).reshape(NC, NS, W_CH, CHK)
    srcw = srcr.reshape(NC, NS, NW3, WIDE)
    dstw = dstr.reshape(NC, NS, NW3 - 2, WIDE)

    xp = jnp.zeros((NP, N_FEAT), x.dtype).at[:N_NODES].set(x)
    z1 = jnp.zeros((NP,), jnp.float32)
    z128 = jnp.zeros((NP, HID), jnp.float32)
    z16 = jnp.zeros((NP, 16), jnp.float32)

    deg_parts = _sc_deg_kernel(dstw, z1)

    p1, dinv, dege = _tc_a(deg_parts, xp, W1)

    agg1 = _sc_agg(p1, srcr, dstr, z128, HID)
    p2 = _tc_b(agg1, p1, dinv, b1.reshape(1, HID), W2)
    agg2 = _sc_agg(p2, srcr, dstr, z128, HID)
    p3 = _tc_c(agg2, p2, dinv, dege, b2.reshape(1, HID),
               W3[:HID], W3[HID:], T_feat, T_adj,
               alpha_param.reshape(1, 1))
    agg3 = _sc_agg16(p3, srcw, dstw, z16)
    b3p = jnp.concatenate([b3, jnp.zeros((8,), jnp.float32)]).reshape(1, 16)
    out = _tc_d(agg3, p3, dinv, b3p)
    return out[:N_NODES, :N_CLS]


# speedup vs baseline: 1.1497x; 1.0258x over previous
import functools

import jax
import jax.numpy as jnp
from jax import lax
from jax.experimental import pallas as pl
from jax.experimental.pallas import tpu as pltpu
from jax.experimental.pallas import tpu_sc as plsc

N_NODES = 10000
N_FEAT = 128
HID = 128
N_T = 16
N_CLS = 8
NP = 10240
BLK = 2048
N_EDGES = 320000
NC = 2
NS = 16
W_CH = 80
CHK = 128
EP = NC * NS * W_CH * CHK
STRIPE = NP // NS

_SC_MESH = dict(core_axis_name="c", subcore_axis_name="s")
WIDE = 1024
NW3 = 12


def _sc_deg_kernel(dstw, zeros1):
    @functools.partial(
        pl.kernel,
        out_type=jax.ShapeDtypeStruct((NC, NP), jnp.float32),
        mesh=plsc.VectorSubcoreMesh(**_SC_MESH),
        compiler_params=pltpu.CompilerParams(use_tc_tiling_on_sc=False),
        scratch_types=[
            pltpu.VMEM((NW3 - 2, WIDE), jnp.int32),
            pltpu.VMEM((WIDE,), jnp.float32),
            pltpu.VMEM_SHARED((NP,), jnp.float32),
        ],
    )
    def body(dstw_hbm, zeros_hbm, out_hbm, dst_v, ones_v, deg_sh):
        c = lax.axis_index("c")
        s = lax.axis_index("s")
        pltpu.sync_copy(dstw_hbm.at[c, s], dst_v)
        pltpu.sync_copy(zeros_hbm.at[pl.ds(s * STRIPE, STRIPE)],
                        deg_sh.at[pl.ds(s * STRIPE, STRIPE)])
        for i in range(WIDE // 16):
            ones_v[pl.ds(i * 16, 16)] = jnp.ones((16,), jnp.float32)
        plsc.subcore_barrier()

        def step(j, carry):
            pltpu.sync_copy(ones_v, deg_sh.at[dst_v.at[j]], add=True)
            return carry

        lax.fori_loop(0, NW3 - 2, step, 0)
        plsc.subcore_barrier()
        pltpu.sync_copy(deg_sh.at[pl.ds(s * STRIPE, STRIPE)],
                        out_hbm.at[c, pl.ds(s * STRIPE, STRIPE)])

    return body(dstw, zeros1)


def _sc_agg(p, srcr, dstr, zeros, D):
    H = W_CH // 2

    @functools.partial(
        pl.kernel,
        out_type=jax.ShapeDtypeStruct((NC, NP, D), jnp.float32),
        mesh=plsc.VectorSubcoreMesh(**_SC_MESH),
        scratch_types=[
            pltpu.VMEM((H + 8, CHK), jnp.int32),
            pltpu.VMEM((H, CHK), jnp.int32),
            pltpu.VMEM((CHK, D), jnp.float32),
            pltpu.VMEM((CHK, D), jnp.float32),
            pltpu.VMEM_SHARED((NP, D), jnp.float32),
            pltpu.SemaphoreType.DMA,
            pltpu.SemaphoreType.DMA,
        ],
    )
    def body(p_hbm, srcr_hbm, dstr_hbm, zeros_hbm, out_hbm,
             src_v, dst_v, rows0, rows1, acc_sh, sem0, sem1):
        c = lax.axis_index("c")
        s = lax.axis_index("s")
        pltpu.sync_copy(zeros_hbm.at[pl.ds(s * STRIPE, STRIPE)],
                        acc_sh.at[pl.ds(s * STRIPE, STRIPE)])
        plsc.subcore_barrier()
        for ph in range(2):
            base = ph * H
            pltpu.sync_copy(srcr_hbm.at[c, s, pl.ds(base, H + 8)], src_v)
            pltpu.sync_copy(dstr_hbm.at[c, s, pl.ds(base, H)], dst_v)
            pltpu.async_copy(p_hbm.at[src_v.at[0]], rows0, sem0)
            pltpu.async_copy(p_hbm.at[src_v.at[1]], rows1, sem1)

            def step(jj, carry):
                j = 2 * jj
                pltpu.make_async_copy(p_hbm.at[src_v.at[0]], rows0, sem0).wait()
                pltpu.sync_copy(rows0, acc_sh.at[dst_v.at[j]], add=True)
                pltpu.async_copy(p_hbm.at[src_v.at[j + 2]], rows0, sem0)
                pltpu.make_async_copy(p_hbm.at[src_v.at[1]], rows1, sem1).wait()
                pltpu.sync_copy(rows1, acc_sh.at[dst_v.at[j + 1]], add=True)
                pltpu.async_copy(p_hbm.at[src_v.at[j + 3]], rows1, sem1)
                return carry

            lax.fori_loop(0, H // 2, step, 0)
            pltpu.make_async_copy(p_hbm.at[src_v.at[0]], rows0, sem0).wait()
            pltpu.make_async_copy(p_hbm.at[src_v.at[1]], rows1, sem1).wait()
        plsc.subcore_barrier()
        pltpu.sync_copy(acc_sh.at[pl.ds(s * STRIPE, STRIPE)],
                        out_hbm.at[c, pl.ds(s * STRIPE, STRIPE)])

    return body(p, srcr, dstr, zeros)


def _sc_agg16(p, srcw, dstw, zeros):
    D = 16

    @functools.partial(
        pl.kernel,
        out_type=jax.ShapeDtypeStruct((NC, NP, D), jnp.float32),
        mesh=plsc.VectorSubcoreMesh(**_SC_MESH),
        compiler_params=pltpu.CompilerParams(use_tc_tiling_on_sc=False),
        scratch_types=[
            pltpu.VMEM((NW3, WIDE), jnp.int32),
            pltpu.VMEM((NW3 - 2, WIDE), jnp.int32),
            pltpu.VMEM((WIDE, D), jnp.float32),
            pltpu.VMEM((WIDE, D), jnp.float32),
            pltpu.VMEM_SHARED((NP, D), jnp.float32),
            pltpu.SemaphoreType.DMA,
            pltpu.SemaphoreType.DMA,
        ],
    )
    def body(p_hbm, srcw_hbm, dstw_hbm, zeros_hbm, out_hbm,
             src_v, dst_v, rows0, rows1, acc_sh, sem0, sem1):
        c = lax.axis_index("c")
        s = lax.axis_index("s")
        pltpu.sync_copy(srcw_hbm.at[c, s], src_v)
        pltpu.sync_copy(dstw_hbm.at[c, s], dst_v)
        pltpu.sync_copy(zeros_hbm.at[pl.ds(s * STRIPE, STRIPE)],
                        acc_sh.at[pl.ds(s * STRIPE, STRIPE)])
        plsc.subcore_barrier()
        pltpu.async_copy(p_hbm.at[src_v.at[0]], rows0, sem0)
        pltpu.async_copy(p_hbm.at[src_v.at[1]], rows1, sem1)

        def step(jj, carry):
            j = 2 * jj
            pltpu.make_async_copy(p_hbm.at[src_v.at[0]], rows0, sem0).wait()
            pltpu.sync_copy(rows0, acc_sh.at[dst_v.at[j]], add=True)
            pltpu.async_copy(p_hbm.at[src_v.at[j + 2]], rows0, sem0)
            pltpu.make_async_copy(p_hbm.at[src_v.at[1]], rows1, sem1).wait()
            pltpu.sync_copy(rows1, acc_sh.at[dst_v.at[j + 1]], add=True)
            pltpu.async_copy(p_hbm.at[src_v.at[j + 3]], rows1, sem1)
            return carry

        lax.fori_loop(0, (NW3 - 2) // 2, step, 0)
        pltpu.make_async_copy(p_hbm.at[src_v.at[0]], rows0, sem0).wait()
        pltpu.make_async_copy(p_hbm.at[src_v.at[1]], rows1, sem1).wait()
        plsc.subcore_barrier()
        pltpu.sync_copy(acc_sh.at[pl.ds(s * STRIPE, STRIPE)],
                        out_hbm.at[c, pl.ds(s * STRIPE, STRIPE)])

    return body(p, srcw, dstw, zeros)


def _tc_a_body(deg_ref, x_ref, w1_ref, p1_ref, dinv_ref, dege_ref):
    deg_e = deg_ref[0, :] + deg_ref[1, :]
    dinv = lax.rsqrt(deg_e + 1.0)
    h = jnp.dot(x_ref[...], w1_ref[...], preferred_element_type=jnp.float32)
    p1_ref[...] = h * dinv[:, None]
    dinv_ref[...] = dinv[:, None]
    dege_ref[...] = deg_e[:, None]


def _tc_b_body(agg_ref, p_ref, dinv_ref, b_ref, w_ref, out_ref):
    dinv = dinv_ref[...]
    agg = agg_ref[0] + agg_ref[1]
    h = jnp.maximum(dinv * (agg + p_ref[...]) + b_ref[...], 0.0)
    out_ref[...] = jnp.dot(h, w_ref[...], preferred_element_type=jnp.float32) * dinv


def _tc_c_body(agg_ref, p_ref, dinv_ref, dege_ref, b2_ref, w3a_ref, w3b_ref,
               tfeat_ref, tadj_ref, alpha_ref, p3_ref):
    dinv = dinv_ref[...]
    agg = agg_ref[0] + agg_ref[1]
    h2 = jnp.maximum(dinv * (agg + p_ref[...]) + b2_ref[...], 0.0)
    tfeat = tfeat_ref[...]
    tadj = tadj_ref[...]
    t_sq = jnp.mean(jnp.sum(tfeat * tfeat, axis=2), axis=1)
    t_mean = jnp.mean(tfeat, axis=1)
    t_deg = jnp.mean(jnp.sum(tadj, axis=2), axis=1)
    alpha = 1.0 / (1.0 + jnp.exp(-alpha_ref[0, 0]))
    x_sq = jnp.sum(h2 * h2, axis=1)
    cross = lax.dot_general(h2, t_mean, (((1,), (1,)), ((), ())),
                            preferred_element_type=jnp.float32)
    c_feat = x_sq[:, None] + t_sq[None, :] - 2.0 * cross
    dege = dege_ref[...]
    c_struct = (dege - t_deg[None, :]) ** 2
    y = alpha * c_feat + (1.0 - alpha) * c_struct
    u = (jnp.dot(h2, w3a_ref[...], preferred_element_type=jnp.float32)
         + jnp.dot(y, w3b_ref[...], preferred_element_type=jnp.float32))
    p3_ref[...] = jnp.concatenate([u, jnp.zeros_like(u)], axis=1) * dinv


def _tc_d_body(agg_ref, p3_ref, dinv_ref, b3_ref, out_ref):
    agg = agg_ref[0] + agg_ref[1]
    out_ref[...] = dinv_ref[...] * (agg + p3_ref[...]) + b3_ref[...]


def _row_spec(cols):
    return pl.BlockSpec((BLK, cols), lambda i: (i, 0))


def _part_spec(cols):
    return pl.BlockSpec((2, BLK, cols), lambda i: (0, i, 0))


def _full_spec(*shape):
    return pl.BlockSpec(shape, lambda i: (0,) * len(shape))


def _tc_a(deg_parts, x, W1):
    return pl.pallas_call(
        _tc_a_body,
        grid=(NP // BLK,),
        in_specs=[pl.BlockSpec((2, BLK), lambda i: (0, i)),
                  _row_spec(N_FEAT), _full_spec(N_FEAT, HID)],
        out_specs=[_row_spec(HID), _row_spec(1), _row_spec(1)],
        out_shape=[jax.ShapeDtypeStruct((NP, HID), jnp.float32),
                   jax.ShapeDtypeStruct((NP, 1), jnp.float32),
                   jax.ShapeDtypeStruct((NP, 1), jnp.float32)],
    )(deg_parts, x, W1)


def _tc_b(agg, p, dinv, b, W):
    return pl.pallas_call(
        _tc_b_body,
        grid=(NP // BLK,),
        in_specs=[_part_spec(HID), _row_spec(HID), _row_spec(1),
                  _full_spec(1, HID), _full_spec(HID, HID)],
        out_specs=_row_spec(HID),
        out_shape=jax.ShapeDtypeStruct((NP, HID), jnp.float32),
    )(agg, p, dinv, b, W)


def _tc_c(agg, p, dinv, dege, b2, W3a, W3b, T_feat, T_adj, alpha):
    return pl.pallas_call(
        _tc_c_body,
        grid=(NP // BLK,),
        in_specs=[_part_spec(HID), _row_spec(HID), _row_spec(1), _row_spec(1),
                  _full_spec(1, HID), _full_spec(HID, N_CLS),
                  _full_spec(N_T, N_CLS), _full_spec(N_T, 16, HID),
                  _full_spec(N_T, 16, 16), _full_spec(1, 1)],
        out_specs=_row_spec(16),
        out_shape=jax.ShapeDtypeStruct((NP, 16), jnp.float32),
    )(agg, p, dinv, dege, b2, W3a, W3b, T_feat, T_adj, alpha)


def _tc_d(agg, p3, dinv, b3):
    return pl.pallas_call(
        _tc_d_body,
        grid=(NP // BLK,),
        in_specs=[_part_spec(16), _row_spec(16), _row_spec(1), _full_spec(1, 16)],
        out_specs=_row_spec(16),
        out_shape=jax.ShapeDtypeStruct((NP, 16), jnp.float32),
    )(agg, p3, dinv, b3)


def kernel(x, edge_index, W1, b1, W2, b2, W3, b3, T_feat, T_adj, alpha_param):
    src = edge_index[0]
    dst = edge_index[1]
    npad = EP - N_EDGES
    pad_src = (jnp.arange(npad, dtype=jnp.int32) % 256)
    pad_dst = N_NODES + (jnp.arange(npad, dtype=jnp.int32) % 240)
    src3 = jnp.concatenate([src, pad_src]).reshape(NC, NS, W_CH, CHK)
    dummy = jnp.broadcast_to(
        (jnp.arange(CHK, dtype=jnp.int32) * 64) % N_NODES, (NC, NS, 16, CHK))
    srcr = jnp.concatenate([src3, dummy], axis=2)
    dstr = jnp.concatenate([dst, pad_dst]).reshape(NC, NS, W_CH, CHK)
    srcw = srcr.reshape(NC, NS, NW3, WIDE)
    dstw = dstr.reshape(NC, NS, NW3 - 2, WIDE)

    xp = jnp.zeros((NP, N_FEAT), x.dtype).at[:N_NODES].set(x)
    z1 = jnp.zeros((NP,), jnp.float32)
    z128 = jnp.zeros((NP, HID), jnp.float32)
    z16 = jnp.zeros((NP, 16), jnp.float32)

    deg_parts = _sc_deg_kernel(dstw, z1)

    p1, dinv, dege = _tc_a(deg_parts, xp, W1)

    agg1 = _sc_agg(p1, srcr, dstr, z128, HID)
    p2 = _tc_b(agg1, p1, dinv, b1.reshape(1, HID), W2)
    agg2 = _sc_agg(p2, srcr, dstr, z128, HID)
    p3 = _tc_c(agg2, p2, dinv, dege, b2.reshape(1, HID),
               W3[:HID], W3[HID:], T_feat, T_adj,
               alpha_param.reshape(1, 1))
    agg3 = _sc_agg16(p3, srcw, dstw, z16)
    b3p = jnp.concatenate([b3, jnp.zeros((8,), jnp.float32)]).reshape(1, 16)
    out = _tc_d(agg3, p3, dinv, b3p)
    return out[:N_NODES, :N_CLS]


# speedup vs baseline: 1.1616x; 1.0104x over previous
import functools

import jax
import jax.numpy as jnp
from jax import lax
from jax.experimental import pallas as pl
from jax.experimental.pallas import tpu as pltpu
from jax.experimental.pallas import tpu_sc as plsc

N_NODES = 10000
N_FEAT = 128
HID = 128
N_T = 16
N_CLS = 8
NP = 10240
BLK = 5120
N_EDGES = 320000
NC = 2
NS = 16
W_CH = 80
CHK = 128
EP = NC * NS * W_CH * CHK
STRIPE = NP // NS

_SC_MESH = dict(core_axis_name="c", subcore_axis_name="s")
WIDE = 1024
NW3 = 12


def _sc_deg_kernel(dstw, zeros1):
    @functools.partial(
        pl.kernel,
        out_type=jax.ShapeDtypeStruct((NC, NP), jnp.float32),
        mesh=plsc.VectorSubcoreMesh(**_SC_MESH),
        compiler_params=pltpu.CompilerParams(use_tc_tiling_on_sc=False),
        scratch_types=[
            pltpu.VMEM((NW3 - 2, WIDE), jnp.int32),
            pltpu.VMEM((WIDE,), jnp.float32),
            pltpu.VMEM_SHARED((NP,), jnp.float32),
        ],
    )
    def body(dstw_hbm, zeros_hbm, out_hbm, dst_v, ones_v, deg_sh):
        c = lax.axis_index("c")
        s = lax.axis_index("s")
        pltpu.sync_copy(dstw_hbm.at[c, s], dst_v)
        pltpu.sync_copy(zeros_hbm.at[pl.ds(s * STRIPE, STRIPE)],
                        deg_sh.at[pl.ds(s * STRIPE, STRIPE)])
        for i in range(WIDE // 16):
            ones_v[pl.ds(i * 16, 16)] = jnp.ones((16,), jnp.float32)
        plsc.subcore_barrier()

        def step(j, carry):
            pltpu.sync_copy(ones_v, deg_sh.at[dst_v.at[j]], add=True)
            return carry

        lax.fori_loop(0, NW3 - 2, step, 0)
        plsc.subcore_barrier()
        pltpu.sync_copy(deg_sh.at[pl.ds(s * STRIPE, STRIPE)],
                        out_hbm.at[c, pl.ds(s * STRIPE, STRIPE)])

    return body(dstw, zeros1)


def _sc_agg(p, srcr, dstr, zeros, D):
    H = W_CH // 2

    @functools.partial(
        pl.kernel,
        out_type=jax.ShapeDtypeStruct((NC, NP, D), jnp.float32),
        mesh=plsc.VectorSubcoreMesh(**_SC_MESH),
        scratch_types=[
            pltpu.VMEM((H + 8, CHK), jnp.int32),
            pltpu.VMEM((H, CHK), jnp.int32),
            pltpu.VMEM((CHK, D), jnp.float32),
            pltpu.VMEM((CHK, D), jnp.float32),
            pltpu.VMEM_SHARED((NP, D), jnp.float32),
            pltpu.SemaphoreType.DMA,
            pltpu.SemaphoreType.DMA,
        ],
    )
    def body(p_hbm, srcr_hbm, dstr_hbm, zeros_hbm, out_hbm,
             src_v, dst_v, rows0, rows1, acc_sh, sem0, sem1):
        c = lax.axis_index("c")
        s = lax.axis_index("s")
        pltpu.sync_copy(zeros_hbm.at[pl.ds(s * STRIPE, STRIPE)],
                        acc_sh.at[pl.ds(s * STRIPE, STRIPE)])
        plsc.subcore_barrier()
        for ph in range(2):
            base = ph * H
            pltpu.sync_copy(srcr_hbm.at[c, s, pl.ds(base, H + 8)], src_v)
            pltpu.sync_copy(dstr_hbm.at[c, s, pl.ds(base, H)], dst_v)
            pltpu.async_copy(p_hbm.at[src_v.at[0]], rows0, sem0)
            pltpu.async_copy(p_hbm.at[src_v.at[1]], rows1, sem1)

            def step(jj, carry):
                j = 2 * jj
                pltpu.make_async_copy(p_hbm.at[src_v.at[0]], rows0, sem0).wait()
                pltpu.sync_copy(rows0, acc_sh.at[dst_v.at[j]], add=True)
                pltpu.async_copy(p_hbm.at[src_v.at[j + 2]], rows0, sem0)
                pltpu.make_async_copy(p_hbm.at[src_v.at[1]], rows1, sem1).wait()
                pltpu.sync_copy(rows1, acc_sh.at[dst_v.at[j + 1]], add=True)
                pltpu.async_copy(p_hbm.at[src_v.at[j + 3]], rows1, sem1)
                return carry

            lax.fori_loop(0, H // 2, step, 0)
            pltpu.make_async_copy(p_hbm.at[src_v.at[0]], rows0, sem0).wait()
            pltpu.make_async_copy(p_hbm.at[src_v.at[1]], rows1, sem1).wait()
        plsc.subcore_barrier()
        pltpu.sync_copy(acc_sh.at[pl.ds(s * STRIPE, STRIPE)],
                        out_hbm.at[c, pl.ds(s * STRIPE, STRIPE)])

    return body(p, srcr, dstr, zeros)


def _sc_agg16(p, srcw, dstw, zeros):
    D = 16

    @functools.partial(
        pl.kernel,
        out_type=jax.ShapeDtypeStruct((NC, NP, D), jnp.float32),
        mesh=plsc.VectorSubcoreMesh(**_SC_MESH),
        compiler_params=pltpu.CompilerParams(use_tc_tiling_on_sc=False),
        scratch_types=[
            pltpu.VMEM((NW3, WIDE), jnp.int32),
            pltpu.VMEM((NW3 - 2, WIDE), jnp.int32),
            pltpu.VMEM((WIDE, D), jnp.float32),
            pltpu.VMEM((WIDE, D), jnp.float32),
            pltpu.VMEM_SHARED((NP, D), jnp.float32),
            pltpu.SemaphoreType.DMA,
            pltpu.SemaphoreType.DMA,
        ],
    )
    def body(p_hbm, srcw_hbm, dstw_hbm, zeros_hbm, out_hbm,
             src_v, dst_v, rows0, rows1, acc_sh, sem0, sem1):
        c = lax.axis_index("c")
        s = lax.axis_index("s")
        pltpu.sync_copy(srcw_hbm.at[c, s], src_v)
        pltpu.sync_copy(dstw_hbm.at[c, s], dst_v)
        pltpu.sync_copy(zeros_hbm.at[pl.ds(s * STRIPE, STRIPE)],
                        acc_sh.at[pl.ds(s * STRIPE, STRIPE)])
        plsc.subcore_barrier()
        pltpu.async_copy(p_hbm.at[src_v.at[0]], rows0, sem0)
        pltpu.async_copy(p_hbm.at[src_v.at[1]], rows1, sem1)

        def step(jj, carry):
            j = 2 * jj
            pltpu.make_async_copy(p_hbm.at[src_v.at[0]], rows0, sem0).wait()
            pltpu.sync_copy(rows0, acc_sh.at[dst_v.at[j]], add=True)
            pltpu.async_copy(p_hbm.at[src_v.at[j + 2]], rows0, sem0)
            pltpu.make_async_copy(p_hbm.at[src_v.at[1]], rows1, sem1).wait()
            pltpu.sync_copy(rows1, acc_sh.at[dst_v.at[j + 1]], add=True)
            pltpu.async_copy(p_hbm.at[src_v.at[j + 3]], rows1, sem1)
            return carry

        lax.fori_loop(0, (NW3 - 2) // 2, step, 0)
        pltpu.make_async_copy(p_hbm.at[src_v.at[0]], rows0, sem0).wait()
        pltpu.make_async_copy(p_hbm.at[src_v.at[1]], rows1, sem1).wait()
        plsc.subcore_barrier()
        pltpu.sync_copy(acc_sh.at[pl.ds(s * STRIPE, STRIPE)],
                        out_hbm.at[c, pl.ds(s * STRIPE, STRIPE)])

    return body(p, srcw, dstw, zeros)


def _tc_a_body(deg_ref, x_ref, w1_ref, p1_ref, dinv_ref, dege_ref):
    deg_e = deg_ref[0, :] + deg_ref[1, :]
    dinv = lax.rsqrt(deg_e + 1.0)
    h = jnp.dot(x_ref[...], w1_ref[...], preferred_element_type=jnp.float32)
    p1_ref[...] = h * dinv[:, None]
    dinv_ref[...] = dinv[:, None]
    dege_ref[...] = deg_e[:, None]


def _tc_b_body(agg_ref, p_ref, dinv_ref, b_ref, w_ref, out_ref):
    dinv = dinv_ref[...]
    agg = agg_ref[0] + agg_ref[1]
    h = jnp.maximum(dinv * (agg + p_ref[...]) + b_ref[...], 0.0)
    out_ref[...] = jnp.dot(h, w_ref[...], preferred_element_type=jnp.float32) * dinv


def _tc_c_body(agg_ref, p_ref, dinv_ref, dege_ref, b2_ref, w3a_ref, w3b_ref,
               tfeat_ref, tadj_ref, alpha_ref, p3_ref):
    dinv = dinv_ref[...]
    agg = agg_ref[0] + agg_ref[1]
    h2 = jnp.maximum(dinv * (agg + p_ref[...]) + b2_ref[...], 0.0)
    tfeat = tfeat_ref[...]
    tadj = tadj_ref[...]
    t_sq = jnp.mean(jnp.sum(tfeat * tfeat, axis=2), axis=1)
    t_mean = jnp.mean(tfeat, axis=1)
    t_deg = jnp.mean(jnp.sum(tadj, axis=2), axis=1)
    alpha = 1.0 / (1.0 + jnp.exp(-alpha_ref[0, 0]))
    x_sq = jnp.sum(h2 * h2, axis=1)
    cross = lax.dot_general(h2, t_mean, (((1,), (1,)), ((), ())),
                            preferred_element_type=jnp.float32)
    c_feat = x_sq[:, None] + t_sq[None, :] - 2.0 * cross
    dege = dege_ref[...]
    c_struct = (dege - t_deg[None, :]) ** 2
    y = alpha * c_feat + (1.0 - alpha) * c_struct
    u = (jnp.dot(h2, w3a_ref[...], preferred_element_type=jnp.float32)
         + jnp.dot(y, w3b_ref[...], preferred_element_type=jnp.float32))
    p3_ref[...] = jnp.concatenate([u, jnp.zeros_like(u)], axis=1) * dinv


def _tc_d_body(agg_ref, p3_ref, dinv_ref, b3_ref, out_ref):
    agg = agg_ref[0] + agg_ref[1]
    out_ref[...] = dinv_ref[...] * (agg + p3_ref[...]) + b3_ref[...]


def _row_spec(cols):
    return pl.BlockSpec((BLK, cols), lambda i: (i, 0))


def _part_spec(cols):
    return pl.BlockSpec((2, BLK, cols), lambda i: (0, i, 0))


def _full_spec(*shape):
    return pl.BlockSpec(shape, lambda i: (0,) * len(shape))


def _tc_a(deg_parts, x, W1):
    return pl.pallas_call(
        _tc_a_body,
        grid=(NP // BLK,),
        in_specs=[pl.BlockSpec((2, BLK), lambda i: (0, i)),
                  _row_spec(N_FEAT), _full_spec(N_FEAT, HID)],
        out_specs=[_row_spec(HID), _row_spec(1), _row_spec(1)],
        out_shape=[jax.ShapeDtypeStruct((NP, HID), jnp.float32),
                   jax.ShapeDtypeStruct((NP, 1), jnp.float32),
                   jax.ShapeDtypeStruct((NP, 1), jnp.float32)],
    )(deg_parts, x, W1)


def _tc_b(agg, p, dinv, b, W):
    return pl.pallas_call(
        _tc_b_body,
        grid=(NP // BLK,),
        in_specs=[_part_spec(HID), _row_spec(HID), _row_spec(1),
                  _full_spec(1, HID), _full_spec(HID, HID)],
        out_specs=_row_spec(HID),
        out_shape=jax.ShapeDtypeStruct((NP, HID), jnp.float32),
    )(agg, p, dinv, b, W)


def _tc_c(agg, p, dinv, dege, b2, W3a, W3b, T_feat, T_adj, alpha):
    return pl.pallas_call(
        _tc_c_body,
        grid=(NP // BLK,),
        in_specs=[_part_spec(HID), _row_spec(HID), _row_spec(1), _row_spec(1),
                  _full_spec(1, HID), _full_spec(HID, N_CLS),
                  _full_spec(N_T, N_CLS), _full_spec(N_T, 16, HID),
                  _full_spec(N_T, 16, 16), _full_spec(1, 1)],
        out_specs=_row_spec(16),
        out_shape=jax.ShapeDtypeStruct((NP, 16), jnp.float32),
    )(agg, p, dinv, dege, b2, W3a, W3b, T_feat, T_adj, alpha)


def _tc_d(agg, p3, dinv, b3):
    return pl.pallas_call(
        _tc_d_body,
        grid=(NP // BLK,),
        in_specs=[_part_spec(16), _row_spec(16), _row_spec(1), _full_spec(1, 16)],
        out_specs=_row_spec(16),
        out_shape=jax.ShapeDtypeStruct((NP, 16), jnp.float32),
    )(agg, p3, dinv, b3)


def kernel(x, edge_index, W1, b1, W2, b2, W3, b3, T_feat, T_adj, alpha_param):
    src = edge_index[0]
    dst = edge_index[1]
    npad = EP - N_EDGES
    pad_src = (jnp.arange(npad, dtype=jnp.int32) % 256)
    pad_dst = N_NODES + (jnp.arange(npad, dtype=jnp.int32) % 240)
    src3 = jnp.concatenate([src, pad_src]).reshape(NC, NS, W_CH, CHK)
    dummy = jnp.broadcast_to(
        (jnp.arange(CHK, dtype=jnp.int32) * 64) % N_NODES, (NC, NS, 16, CHK))
    srcr = jnp.concatenate([src3, dummy], axis=2)
    dstr = jnp.concatenate([dst, pad_dst]).reshape(NC, NS, W_CH, CHK)
    srcw = srcr.reshape(NC, NS, NW3, WIDE)
    dstw = dstr.reshape(NC, NS, NW3 - 2, WIDE)

    xp = jnp.zeros((NP, N_FEAT), x.dtype).at[:N_NODES].set(x)
    z1 = jnp.zeros((NP,), jnp.float32)
    z128 = jnp.zeros((NP, HID), jnp.float32)
    z16 = jnp.zeros((NP, 16), jnp.float32)

    deg_parts = _sc_deg_kernel(dstw, z1)

    p1, dinv, dege = _tc_a(deg_parts, xp, W1)

    agg1 = _sc_agg(p1, srcr, dstr, z128, HID)
    p2 = _tc_b(agg1, p1, dinv, b1.reshape(1, HID), W2)
    agg2 = _sc_agg(p2, srcr, dstr, z128, HID)
    p3 = _tc_c(agg2, p2, dinv, dege, b2.reshape(1, HID),
               W3[:HID], W3[HID:], T_feat, T_adj,
               alpha_param.reshape(1, 1))
    agg3 = _sc_agg16(p3, srcw, dstw, z16)
    b3p = jnp.concatenate([b3, jnp.zeros((8,), jnp.float32)]).reshape(1, 16)
    out = _tc_d(agg3, p3, dinv, b3p)
    return out[:N_NODES, :N_CLS]
